# baseline (speedup 1.0000x reference)
import functools

import jax
import jax.numpy as jnp
import numpy as np
from jax import lax
from jax.experimental import pallas as pl
from jax.experimental.pallas import tpu as pltpu
from jax.experimental.pallas import tpu_sc as plsc

_NC, _NS, _L = 2, 16, 16
_NW = _NC * _NS


def _knn_kernel(xt_ref, xs_ref, idx_ref, w_ref):
    S = xs_ref.shape[1]
    b = pl.program_id(0)
    xt = xt_ref[0]
    xs = xs_ref[0]
    xt2 = jnp.sum(xt * xt, axis=1, keepdims=True)
    xs2 = jnp.sum(xs * xs, axis=1, keepdims=True)
    dot = jnp.dot(xt, xs.T, preferred_element_type=jnp.float32)
    d = jnp.maximum(xt2 + xs2.T - 2.0 * dot, 0.0)

    inf = jnp.float32(np.inf)
    iota = jax.lax.broadcasted_iota(jnp.int32, d.shape, 1)
    work = d
    idxs, vals = [], []
    for _ in range(3):
        m = jnp.min(work, axis=1, keepdims=True)
        fidx = jnp.min(jnp.where(work == m, iota, S), axis=1, keepdims=True)
        vals.append(m)
        idxs.append(fidx)
        work = jnp.where(iota == fidx, inf, work)
    i1, i2, i3 = idxs
    r1 = 1.0 / (vals[0] + 1e-8)
    r2 = 1.0 / (vals[1] + 1e-8)
    r3 = 1.0 / (vals[2] + 1e-8)
    inv_norm = 1.0 / (r1 + r2 + r3)
    idx_ref[0] = jnp.concatenate([i1, i2, i3], axis=1) + b * S
    NB = i1.shape[0]
    w_ref[0] = jnp.concatenate(
        [jnp.broadcast_to(r1 * inv_norm, (NB, _L)),
         jnp.broadcast_to(r2 * inv_norm, (NB, _L)),
         jnp.broadcast_to(r3 * inv_norm, (NB, _L))], axis=1)


def _sc_interp_body(fs_hbm, idx_hbm, w_hbm, out_hbm, idx_v, w_v,
                    rows0, rows1, out0, out1, gsem0, gsem1, osem0, osem1,
                    *, rows_per_w, n_chunks, ch, cs):
    wid = lax.axis_index("s") * _NC + lax.axis_index("c")
    base = wid * rows_per_w
    nck = 3 * ch
    pltpu.sync_copy(idx_hbm.at[pl.ds(base * 3, rows_per_w * 3)], idx_v)
    pltpu.sync_copy(w_hbm.at[pl.ds(base * 3 * _L, rows_per_w * 3 * _L)], w_v)
    pltpu.async_copy(fs_hbm.at[idx_v.at[pl.ds(0, nck)]], rows0, gsem0)
    pltpu.async_copy(fs_hbm.at[idx_v.at[pl.ds(nck, nck)]], rows1, gsem1)
    last = (n_chunks - 1) * nck

    def do_chunk(i, rows_v, out_v, gsem, osem, wait_out):
        pltpu.make_async_copy(fs_hbm.at[idx_v.at[pl.ds(0, nck)]],
                              rows_v, gsem).wait()
        if wait_out:
            pltpu.make_async_copy(out_v, out_hbm.at[pl.ds(0, ch)], osem).wait()
        for t in range(ch):
            wb = (i * ch + t) * 3 * _L
            w0 = w_v[pl.ds(wb, _L)]
            w1 = w_v[pl.ds(wb + _L, _L)]
            w2 = w_v[pl.ds(wb + 2 * _L, _L)]
            for c in range(cs // _L):
                sl = pl.ds(c * _L, _L)
                out_v[t, sl] = (w0 * rows_v[3 * t, sl]
                                + w1 * rows_v[3 * t + 1, sl]
                                + w2 * rows_v[3 * t + 2, sl])
        pltpu.async_copy(out_v, out_hbm.at[pl.ds(base + i * ch, ch)], osem)
        nxt = jnp.minimum((i + 2) * nck, last)
        pltpu.async_copy(fs_hbm.at[idx_v.at[pl.ds(nxt, nck)]], rows_v, gsem)

    do_chunk(0, rows0, out0, gsem0, osem0, False)
    do_chunk(1, rows1, out1, gsem1, osem1, False)

    def body(j, carry):
        do_chunk(2 * j, rows0, out0, gsem0, osem0, True)
        do_chunk(2 * j + 1, rows1, out1, gsem1, osem1, True)
        return carry

    lax.fori_loop(1, n_chunks // 2, body, 0)
    pltpu.make_async_copy(fs_hbm.at[idx_v.at[pl.ds(0, nck)]], rows0, gsem0).wait()
    pltpu.make_async_copy(fs_hbm.at[idx_v.at[pl.ds(0, nck)]], rows1, gsem1).wait()
    pltpu.make_async_copy(out0, out_hbm.at[pl.ds(0, ch)], osem0).wait()
    pltpu.make_async_copy(out1, out_hbm.at[pl.ds(0, ch)], osem1).wait()


def _make_sc_interp(BN, CS):
    ch = 16
    rows_per_w = BN // _NW
    n_chunks = rows_per_w // ch
    mesh = plsc.VectorSubcoreMesh(core_axis_name="c", subcore_axis_name="s",
                                  num_cores=_NC, num_subcores=_NS)
    return pl.kernel(
        functools.partial(_sc_interp_body, rows_per_w=rows_per_w,
                          n_chunks=n_chunks, ch=ch, cs=CS),
        mesh=mesh,
        out_type=jax.ShapeDtypeStruct((BN, CS), jnp.float32),
        scratch_types=[
            pltpu.VMEM((3 * rows_per_w,), jnp.int32),
            pltpu.VMEM((rows_per_w * 3 * _L,), jnp.float32),
            pltpu.VMEM((3 * ch, CS), jnp.float32),
            pltpu.VMEM((3 * ch, CS), jnp.float32),
            pltpu.VMEM((ch, CS), jnp.float32),
            pltpu.VMEM((ch, CS), jnp.float32),
            pltpu.SemaphoreType.DMA,
            pltpu.SemaphoreType.DMA,
            pltpu.SemaphoreType.DMA,
            pltpu.SemaphoreType.DMA,
        ],
    )


def _l1_kernel(ft_ref, interp_ref, w1t_ref, x1_ref, stats_ref):
    CT = ft_ref.shape[1]
    x1 = (jnp.dot(ft_ref[...], w1t_ref[:CT, :], preferred_element_type=jnp.float32)
          + jnp.dot(interp_ref[...], w1t_ref[CT:, :], preferred_element_type=jnp.float32))
    x1_ref[...] = x1

    @pl.when(pl.program_id(0) == 0)
    def _init():
        stats_ref[...] = jnp.zeros_like(stats_ref)

    stats_ref[0, :] += jnp.sum(x1, axis=0)
    stats_ref[1, :] += jnp.sum(x1 * x1, axis=0)


def _mlp_kernel(x_ref, a_ref, c_ref, w2t_ref, x2_ref, stats_ref):
    y = jnp.maximum(x_ref[...] * a_ref[...] + c_ref[...], 0.0)
    x2 = jnp.dot(y, w2t_ref[...], preferred_element_type=jnp.float32)
    x2_ref[...] = x2

    @pl.when(pl.program_id(0) == 0)
    def _init():
        stats_ref[...] = jnp.zeros_like(stats_ref)

    stats_ref[0, :] += jnp.sum(x2, axis=0)
    stats_ref[1, :] += jnp.sum(x2 * x2, axis=0)


def _bnrelu_kernel(x_ref, a_ref, c_ref, o_ref):
    o_ref[...] = jnp.maximum(x_ref[...] * a_ref[...] + c_ref[...], 0.0)


def _bn_coeffs(stats, g, b, cnt):
    mean = stats[0] / cnt
    var = stats[1] / cnt - mean * mean
    rstd = jax.lax.rsqrt(var + 1e-5)
    a = (g * rstd)[None, :]
    c = (b - g * rstd * mean)[None, :]
    return a, c


def kernel(xyz_target, xyz_source, feat_target, feat_source, W1, g1, b1, W2, g2, b2):
    B, N, _ = xyz_target.shape
    S = xyz_source.shape[1]
    CT = feat_target.shape[2]
    CS = feat_source.shape[2]
    M1 = W1.shape[0]
    M2 = W2.shape[0]
    NB = min(512, N)
    gN = N // NB
    BN = B * N
    W1t = W1.T
    fs_flat = feat_source.reshape(B * S, CS)

    idxg, wn = pl.pallas_call(
        _knn_kernel,
        grid=(B, gN),
        in_specs=[
            pl.BlockSpec((1, NB, 3), lambda b, n: (b, n, 0)),
            pl.BlockSpec((1, S, 3), lambda b, n: (b, 0, 0)),
        ],
        out_specs=[
            pl.BlockSpec((1, NB, 3), lambda b, n: (b, n, 0)),
            pl.BlockSpec((1, NB, 3 * _L), lambda b, n: (b, n, 0)),
        ],
        out_shape=[
            jax.ShapeDtypeStruct((B, N, 3), jnp.int32),
            jax.ShapeDtypeStruct((B, N, 3 * _L), jnp.float32),
        ],
    )(xyz_target, xyz_source)

    interp = _make_sc_interp(BN, CS)(
        fs_flat,
        idxg.reshape(BN * 3),
        wn.reshape(BN * 3 * _L),
    )

    NB2 = min(2048, BN)
    g2n = BN // NB2
    x1, stats1 = pl.pallas_call(
        _l1_kernel,
        grid=(g2n,),
        in_specs=[
            pl.BlockSpec((NB2, CT), lambda i: (i, 0)),
            pl.BlockSpec((NB2, CS), lambda i: (i, 0)),
            pl.BlockSpec((CT + CS, M1), lambda i: (0, 0)),
        ],
        out_specs=[
            pl.BlockSpec((NB2, M1), lambda i: (i, 0)),
            pl.BlockSpec((2, M1), lambda i: (0, 0)),
        ],
        out_shape=[
            jax.ShapeDtypeStruct((BN, M1), jnp.float32),
            jax.ShapeDtypeStruct((2, M1), jnp.float32),
        ],
    )(feat_target.reshape(BN, CT), interp, W1t)

    cnt = jnp.float32(BN)
    a1, c1 = _bn_coeffs(stats1, g1, b1, cnt)

    x2, stats2 = pl.pallas_call(
        _mlp_kernel,
        grid=(g2n,),
        in_specs=[
            pl.BlockSpec((NB2, M1), lambda i: (i, 0)),
            pl.BlockSpec((1, M1), lambda i: (0, 0)),
            pl.BlockSpec((1, M1), lambda i: (0, 0)),
            pl.BlockSpec((M1, M2), lambda i: (0, 0)),
        ],
        out_specs=[
            pl.BlockSpec((NB2, M2), lambda i: (i, 0)),
            pl.BlockSpec((2, M2), lambda i: (0, 0)),
        ],
        out_shape=[
            jax.ShapeDtypeStruct((BN, M2), jnp.float32),
            jax.ShapeDtypeStruct((2, M2), jnp.float32),
        ],
    )(x1, a1, c1, W2.T)

    a2, c2 = _bn_coeffs(stats2, g2, b2, cnt)

    out = pl.pallas_call(
        _bnrelu_kernel,
        grid=(g2n,),
        in_specs=[
            pl.BlockSpec((NB2, M2), lambda i: (i, 0)),
            pl.BlockSpec((1, M2), lambda i: (0, 0)),
            pl.BlockSpec((1, M2), lambda i: (0, 0)),
        ],
        out_specs=pl.BlockSpec((NB2, M2), lambda i: (i, 0)),
        out_shape=jax.ShapeDtypeStruct((BN, M2), jnp.float32),
    )(x2, a2, c2)
    return out.reshape(B, N, M2)


# speedup vs baseline: 1.0424x; 1.0424x over previous
import functools

import jax
import jax.numpy as jnp
import numpy as np
from jax import lax
from jax.experimental import pallas as pl
from jax.experimental.pallas import tpu as pltpu
from jax.experimental.pallas import tpu_sc as plsc

_NC, _NS, _L = 2, 16, 16
_NW = _NC * _NS


def _knn_kernel(xt_ref, xs_ref, idx_ref, w_ref):
    S = xs_ref.shape[1]
    b = pl.program_id(0)
    xt = xt_ref[0]
    xs = xs_ref[0]
    xt2 = jnp.sum(xt * xt, axis=1, keepdims=True)
    xs2 = jnp.sum(xs * xs, axis=1, keepdims=True)
    dot = jnp.dot(xt, xs.T, preferred_element_type=jnp.float32)
    d = jnp.maximum(xt2 + xs2.T - 2.0 * dot, 0.0)

    inf = jnp.float32(np.inf)
    iota = jax.lax.broadcasted_iota(jnp.int32, d.shape, 1)
    work = d
    idxs, vals = [], []
    for _ in range(3):
        m = jnp.min(work, axis=1, keepdims=True)
        fidx = jnp.min(jnp.where(work == m, iota, S), axis=1, keepdims=True)
        vals.append(m)
        idxs.append(fidx)
        work = jnp.where(iota == fidx, inf, work)
    i1, i2, i3 = idxs
    r1 = 1.0 / (vals[0] + 1e-8)
    r2 = 1.0 / (vals[1] + 1e-8)
    r3 = 1.0 / (vals[2] + 1e-8)
    inv_norm = 1.0 / (r1 + r2 + r3)
    idx_ref[0] = jnp.concatenate([i1, i2, i3], axis=1) + b * S
    NB = i1.shape[0]
    w_ref[0] = jnp.concatenate(
        [jnp.broadcast_to(r1 * inv_norm, (NB, _L)),
         jnp.broadcast_to(r2 * inv_norm, (NB, _L)),
         jnp.broadcast_to(r3 * inv_norm, (NB, _L))], axis=1)


def _sc_interp_body(fs_hbm, idx_hbm, w_hbm, out_hbm, idx_v, w_v,
                    rows0, rows1, out0, out1, gsem0, gsem1, osem0, osem1,
                    *, rows_per_w, n_chunks, ch, cs):
    wid = lax.axis_index("s") * _NC + lax.axis_index("c")
    base = wid * rows_per_w
    nck = 3 * ch
    pltpu.sync_copy(idx_hbm.at[pl.ds(base * 3, rows_per_w * 3)], idx_v)
    pltpu.sync_copy(w_hbm.at[pl.ds(base * 3 * _L, rows_per_w * 3 * _L)], w_v)
    pltpu.async_copy(fs_hbm.at[idx_v.at[pl.ds(0, nck)]], rows0, gsem0)
    pltpu.async_copy(fs_hbm.at[idx_v.at[pl.ds(nck, nck)]], rows1, gsem1)
    last = (n_chunks - 1) * nck

    def do_chunk(i, rows_v, out_v, gsem, osem, wait_out):
        pltpu.make_async_copy(fs_hbm.at[idx_v.at[pl.ds(0, nck)]],
                              rows_v, gsem).wait()
        if wait_out:
            pltpu.make_async_copy(out_v, out_hbm.at[pl.ds(0, ch)], osem).wait()
        for t in range(ch):
            wb = (i * ch + t) * 3 * _L
            w0 = w_v[pl.ds(wb, _L)]
            w1 = w_v[pl.ds(wb + _L, _L)]
            w2 = w_v[pl.ds(wb + 2 * _L, _L)]
            for c in range(cs // _L):
                sl = pl.ds(c * _L, _L)
                out_v[t, sl] = (w0 * rows_v[3 * t, sl]
                                + w1 * rows_v[3 * t + 1, sl]
                                + w2 * rows_v[3 * t + 2, sl])
        pltpu.async_copy(out_v, out_hbm.at[pl.ds(base + i * ch, ch)], osem)
        nxt = jnp.minimum((i + 2) * nck, last)
        pltpu.async_copy(fs_hbm.at[idx_v.at[pl.ds(nxt, nck)]], rows_v, gsem)

    do_chunk(0, rows0, out0, gsem0, osem0, False)
    do_chunk(1, rows1, out1, gsem1, osem1, False)

    def body(j, carry):
        do_chunk(2 * j, rows0, out0, gsem0, osem0, True)
        do_chunk(2 * j + 1, rows1, out1, gsem1, osem1, True)
        return carry

    lax.fori_loop(1, n_chunks // 2, body, 0)
    pltpu.make_async_copy(fs_hbm.at[idx_v.at[pl.ds(0, nck)]], rows0, gsem0).wait()
    pltpu.make_async_copy(fs_hbm.at[idx_v.at[pl.ds(0, nck)]], rows1, gsem1).wait()
    pltpu.make_async_copy(out0, out_hbm.at[pl.ds(0, ch)], osem0).wait()
    pltpu.make_async_copy(out1, out_hbm.at[pl.ds(0, ch)], osem1).wait()


def _make_sc_interp(BN, CS):
    ch = 16
    rows_per_w = BN // _NW
    n_chunks = rows_per_w // ch
    mesh = plsc.VectorSubcoreMesh(core_axis_name="c", subcore_axis_name="s",
                                  num_cores=_NC, num_subcores=_NS)
    return pl.kernel(
        functools.partial(_sc_interp_body, rows_per_w=rows_per_w,
                          n_chunks=n_chunks, ch=ch, cs=CS),
        mesh=mesh,
        out_type=jax.ShapeDtypeStruct((BN, CS), jnp.float32),
        scratch_types=[
            pltpu.VMEM((3 * rows_per_w,), jnp.int32),
            pltpu.VMEM((rows_per_w * 3 * _L,), jnp.float32),
            pltpu.VMEM((3 * ch, CS), jnp.float32),
            pltpu.VMEM((3 * ch, CS), jnp.float32),
            pltpu.VMEM((ch, CS), jnp.float32),
            pltpu.VMEM((ch, CS), jnp.float32),
            pltpu.SemaphoreType.DMA,
            pltpu.SemaphoreType.DMA,
            pltpu.SemaphoreType.DMA,
            pltpu.SemaphoreType.DMA,
        ],
    )


def _l1_kernel(ft_ref, interp_ref, w1t_ref, x1_ref, stats_ref):
    CT = ft_ref.shape[1]
    x1 = (jnp.dot(ft_ref[...], w1t_ref[:CT, :], preferred_element_type=jnp.float32)
          + jnp.dot(interp_ref[...], w1t_ref[CT:, :], preferred_element_type=jnp.float32))
    x1_ref[...] = x1

    @pl.when(pl.program_id(0) == 0)
    def _init():
        stats_ref[...] = jnp.zeros_like(stats_ref)

    stats_ref[0, :] += jnp.sum(x1, axis=0)
    stats_ref[1, :] += jnp.sum(x1 * x1, axis=0)


def _mlp_kernel(x_ref, a_ref, c_ref, w2t_ref, x2_ref, stats_ref):
    y = jnp.maximum(x_ref[...] * a_ref[...] + c_ref[...], 0.0)
    x2 = jnp.dot(y, w2t_ref[...], preferred_element_type=jnp.float32)
    x2_ref[...] = x2

    @pl.when(pl.program_id(0) == 0)
    def _init():
        stats_ref[...] = jnp.zeros_like(stats_ref)

    stats_ref[0, :] += jnp.sum(x2, axis=0)
    stats_ref[1, :] += jnp.sum(x2 * x2, axis=0)


def _bnrelu_kernel(x_ref, a_ref, c_ref, o_ref):
    o_ref[...] = jnp.maximum(x_ref[...] * a_ref[...] + c_ref[...], 0.0)


def _bn_coeffs(stats, g, b, cnt):
    mean = stats[0] / cnt
    var = stats[1] / cnt - mean * mean
    rstd = jax.lax.rsqrt(var + 1e-5)
    a = (g * rstd)[None, :]
    c = (b - g * rstd * mean)[None, :]
    return a, c


def kernel(xyz_target, xyz_source, feat_target, feat_source, W1, g1, b1, W2, g2, b2):
    B, N, _ = xyz_target.shape
    S = xyz_source.shape[1]
    CT = feat_target.shape[2]
    CS = feat_source.shape[2]
    M1 = W1.shape[0]
    M2 = W2.shape[0]
    NB = min(1024, N)
    gN = N // NB
    BN = B * N
    W1t = W1.T
    fs_flat = feat_source.reshape(B * S, CS)

    idxg, wn = pl.pallas_call(
        _knn_kernel,
        grid=(B, gN),
        in_specs=[
            pl.BlockSpec((1, NB, 3), lambda b, n: (b, n, 0)),
            pl.BlockSpec((1, S, 3), lambda b, n: (b, 0, 0)),
        ],
        out_specs=[
            pl.BlockSpec((1, NB, 3), lambda b, n: (b, n, 0)),
            pl.BlockSpec((1, NB, 3 * _L), lambda b, n: (b, n, 0)),
        ],
        out_shape=[
            jax.ShapeDtypeStruct((B, N, 3), jnp.int32),
            jax.ShapeDtypeStruct((B, N, 3 * _L), jnp.float32),
        ],
    )(xyz_target, xyz_source)

    interp = _make_sc_interp(BN, CS)(
        fs_flat,
        idxg.reshape(BN * 3),
        wn.reshape(BN * 3 * _L),
    )

    NB2 = min(2048, BN)
    g2n = BN // NB2
    x1, stats1 = pl.pallas_call(
        _l1_kernel,
        grid=(g2n,),
        in_specs=[
            pl.BlockSpec((NB2, CT), lambda i: (i, 0)),
            pl.BlockSpec((NB2, CS), lambda i: (i, 0)),
            pl.BlockSpec((CT + CS, M1), lambda i: (0, 0)),
        ],
        out_specs=[
            pl.BlockSpec((NB2, M1), lambda i: (i, 0)),
            pl.BlockSpec((2, M1), lambda i: (0, 0)),
        ],
        out_shape=[
            jax.ShapeDtypeStruct((BN, M1), jnp.float32),
            jax.ShapeDtypeStruct((2, M1), jnp.float32),
        ],
    )(feat_target.reshape(BN, CT), interp, W1t)

    cnt = jnp.float32(BN)
    a1, c1 = _bn_coeffs(stats1, g1, b1, cnt)

    x2, stats2 = pl.pallas_call(
        _mlp_kernel,
        grid=(g2n,),
        in_specs=[
            pl.BlockSpec((NB2, M1), lambda i: (i, 0)),
            pl.BlockSpec((1, M1), lambda i: (0, 0)),
            pl.BlockSpec((1, M1), lambda i: (0, 0)),
            pl.BlockSpec((M1, M2), lambda i: (0, 0)),
        ],
        out_specs=[
            pl.BlockSpec((NB2, M2), lambda i: (i, 0)),
            pl.BlockSpec((2, M2), lambda i: (0, 0)),
        ],
        out_shape=[
            jax.ShapeDtypeStruct((BN, M2), jnp.float32),
            jax.ShapeDtypeStruct((2, M2), jnp.float32),
        ],
    )(x1, a1, c1, W2.T)

    a2, c2 = _bn_coeffs(stats2, g2, b2, cnt)

    out = pl.pallas_call(
        _bnrelu_kernel,
        grid=(g2n,),
        in_specs=[
            pl.BlockSpec((NB2, M2), lambda i: (i, 0)),
            pl.BlockSpec((1, M2), lambda i: (0, 0)),
            pl.BlockSpec((1, M2), lambda i: (0, 0)),
        ],
        out_specs=pl.BlockSpec((NB2, M2), lambda i: (i, 0)),
        out_shape=jax.ShapeDtypeStruct((BN, M2), jnp.float32),
    )(x2, a2, c2)
    return out.reshape(B, N, M2)


# speedup vs baseline: 1.0610x; 1.0178x over previous
import functools

import jax
import jax.numpy as jnp
import numpy as np
from jax import lax
from jax.experimental import pallas as pl
from jax.experimental.pallas import tpu as pltpu
from jax.experimental.pallas import tpu_sc as plsc

_NC, _NS, _L = 2, 16, 16
_NW = _NC * _NS


def _knn_kernel(xt_ref, xs_ref, idx_ref, w_ref):
    S = xs_ref.shape[1]
    b = pl.program_id(0)
    xt = xt_ref[0]
    xs = xs_ref[0]
    xt2 = jnp.sum(xt * xt, axis=1, keepdims=True)
    xs2 = jnp.sum(xs * xs, axis=1, keepdims=True)
    dot = jnp.dot(xt, xs.T, preferred_element_type=jnp.float32)
    d = jnp.maximum(xt2 + xs2.T - 2.0 * dot, 0.0)

    inf = jnp.float32(np.inf)
    iota = jax.lax.broadcasted_iota(jnp.int32, d.shape, 1)
    work = d
    idxs, vals = [], []
    for _ in range(3):
        m = jnp.min(work, axis=1, keepdims=True)
        fidx = jnp.min(jnp.where(work == m, iota, S), axis=1, keepdims=True)
        vals.append(m)
        idxs.append(fidx)
        work = jnp.where(iota == fidx, inf, work)
    i1, i2, i3 = idxs
    r1 = 1.0 / (vals[0] + 1e-8)
    r2 = 1.0 / (vals[1] + 1e-8)
    r3 = 1.0 / (vals[2] + 1e-8)
    inv_norm = 1.0 / (r1 + r2 + r3)
    idx_ref[0] = jnp.concatenate([i1, i2, i3], axis=1) + b * S
    NB = i1.shape[0]
    w_ref[0] = jnp.concatenate(
        [jnp.broadcast_to(r1 * inv_norm, (NB, _L)),
         jnp.broadcast_to(r2 * inv_norm, (NB, _L)),
         jnp.broadcast_to(r3 * inv_norm, (NB, _L))], axis=1)


def _sc_interp_body(fs_hbm, idx_hbm, w_hbm, out_hbm, idx_v, w_v,
                    rows0, rows1, out0, out1, gsem0, gsem1, osem0, osem1,
                    *, rows_per_w, n_chunks, ch, cs):
    wid = lax.axis_index("s") * _NC + lax.axis_index("c")
    base = wid * rows_per_w
    nck = 3 * ch
    pltpu.sync_copy(idx_hbm.at[pl.ds(base * 3, rows_per_w * 3)], idx_v)
    pltpu.sync_copy(w_hbm.at[pl.ds(base * 3 * _L, rows_per_w * 3 * _L)], w_v)
    pltpu.async_copy(fs_hbm.at[idx_v.at[pl.ds(0, nck)]], rows0, gsem0)
    pltpu.async_copy(fs_hbm.at[idx_v.at[pl.ds(nck, nck)]], rows1, gsem1)
    last = (n_chunks - 1) * nck

    def do_chunk(i, rows_v, out_v, gsem, osem, wait_out):
        pltpu.make_async_copy(fs_hbm.at[idx_v.at[pl.ds(0, nck)]],
                              rows_v, gsem).wait()
        if wait_out:
            pltpu.make_async_copy(out_v, out_hbm.at[pl.ds(0, ch)], osem).wait()
        for t in range(ch):
            wb = (i * ch + t) * 3 * _L
            w0 = w_v[pl.ds(wb, _L)]
            w1 = w_v[pl.ds(wb + _L, _L)]
            w2 = w_v[pl.ds(wb + 2 * _L, _L)]
            for c in range(cs // _L):
                sl = pl.ds(c * _L, _L)
                out_v[t, sl] = (w0 * rows_v[3 * t, sl]
                                + w1 * rows_v[3 * t + 1, sl]
                                + w2 * rows_v[3 * t + 2, sl])
        pltpu.async_copy(out_v, out_hbm.at[pl.ds(base + i * ch, ch)], osem)
        nxt = jnp.minimum((i + 2) * nck, last)
        pltpu.async_copy(fs_hbm.at[idx_v.at[pl.ds(nxt, nck)]], rows_v, gsem)

    do_chunk(0, rows0, out0, gsem0, osem0, False)
    do_chunk(1, rows1, out1, gsem1, osem1, False)

    def body(j, carry):
        do_chunk(2 * j, rows0, out0, gsem0, osem0, True)
        do_chunk(2 * j + 1, rows1, out1, gsem1, osem1, True)
        return carry

    lax.fori_loop(1, n_chunks // 2, body, 0)
    pltpu.make_async_copy(fs_hbm.at[idx_v.at[pl.ds(0, nck)]], rows0, gsem0).wait()
    pltpu.make_async_copy(fs_hbm.at[idx_v.at[pl.ds(0, nck)]], rows1, gsem1).wait()
    pltpu.make_async_copy(out0, out_hbm.at[pl.ds(0, ch)], osem0).wait()
    pltpu.make_async_copy(out1, out_hbm.at[pl.ds(0, ch)], osem1).wait()


def _make_sc_interp(BN, CS):
    ch = 16
    rows_per_w = BN // _NW
    n_chunks = rows_per_w // ch
    mesh = plsc.VectorSubcoreMesh(core_axis_name="c", subcore_axis_name="s",
                                  num_cores=_NC, num_subcores=_NS)
    return pl.kernel(
        functools.partial(_sc_interp_body, rows_per_w=rows_per_w,
                          n_chunks=n_chunks, ch=ch, cs=CS),
        mesh=mesh,
        out_type=jax.ShapeDtypeStruct((BN, CS), jnp.float32),
        scratch_types=[
            pltpu.VMEM((3 * rows_per_w,), jnp.int32),
            pltpu.VMEM((rows_per_w * 3 * _L,), jnp.float32),
            pltpu.VMEM((3 * ch, CS), jnp.float32),
            pltpu.VMEM((3 * ch, CS), jnp.float32),
            pltpu.VMEM((ch, CS), jnp.float32),
            pltpu.VMEM((ch, CS), jnp.float32),
            pltpu.SemaphoreType.DMA,
            pltpu.SemaphoreType.DMA,
            pltpu.SemaphoreType.DMA,
            pltpu.SemaphoreType.DMA,
        ],
    )


def _l1_kernel(ft_ref, interp_ref, w1t_ref, x1_ref, stats_ref):
    CT = ft_ref.shape[1]
    x1 = (jnp.dot(ft_ref[...], w1t_ref[:CT, :], preferred_element_type=jnp.float32)
          + jnp.dot(interp_ref[...], w1t_ref[CT:, :], preferred_element_type=jnp.float32))
    x1_ref[...] = x1

    @pl.when(pl.program_id(0) == 0)
    def _init():
        stats_ref[...] = jnp.zeros_like(stats_ref)

    stats_ref[0, :] += jnp.sum(x1, axis=0)
    stats_ref[1, :] += jnp.sum(x1 * x1, axis=0)


def _mlp_kernel(x_ref, a_ref, c_ref, w2t_ref, x2_ref, stats_ref):
    y = jnp.maximum(x_ref[...] * a_ref[...] + c_ref[...], 0.0)
    x2 = jnp.dot(y, w2t_ref[...], preferred_element_type=jnp.float32)
    x2_ref[...] = x2

    @pl.when(pl.program_id(0) == 0)
    def _init():
        stats_ref[...] = jnp.zeros_like(stats_ref)

    stats_ref[0, :] += jnp.sum(x2, axis=0)
    stats_ref[1, :] += jnp.sum(x2 * x2, axis=0)


def _bnrelu_kernel(x_ref, a_ref, c_ref, o_ref):
    o_ref[...] = jnp.maximum(x_ref[...] * a_ref[...] + c_ref[...], 0.0)


def _bn_coeffs(stats, g, b, cnt):
    mean = stats[0] / cnt
    var = stats[1] / cnt - mean * mean
    rstd = jax.lax.rsqrt(var + 1e-5)
    a = (g * rstd)[None, :]
    c = (b - g * rstd * mean)[None, :]
    return a, c


def kernel(xyz_target, xyz_source, feat_target, feat_source, W1, g1, b1, W2, g2, b2):
    B, N, _ = xyz_target.shape
    S = xyz_source.shape[1]
    CT = feat_target.shape[2]
    CS = feat_source.shape[2]
    M1 = W1.shape[0]
    M2 = W2.shape[0]
    NB = min(2048, N)
    gN = N // NB
    BN = B * N
    W1t = W1.T
    fs_flat = feat_source.reshape(B * S, CS)

    idxg, wn = pl.pallas_call(
        _knn_kernel,
        grid=(B, gN),
        in_specs=[
            pl.BlockSpec((1, NB, 3), lambda b, n: (b, n, 0)),
            pl.BlockSpec((1, S, 3), lambda b, n: (b, 0, 0)),
        ],
        out_specs=[
            pl.BlockSpec((1, NB, 3), lambda b, n: (b, n, 0)),
            pl.BlockSpec((1, NB, 3 * _L), lambda b, n: (b, n, 0)),
        ],
        out_shape=[
            jax.ShapeDtypeStruct((B, N, 3), jnp.int32),
            jax.ShapeDtypeStruct((B, N, 3 * _L), jnp.float32),
        ],
    )(xyz_target, xyz_source)

    interp = _make_sc_interp(BN, CS)(
        fs_flat,
        idxg.reshape(BN * 3),
        wn.reshape(BN * 3 * _L),
    )

    NB2 = min(2048, BN)
    g2n = BN // NB2
    x1, stats1 = pl.pallas_call(
        _l1_kernel,
        grid=(g2n,),
        in_specs=[
            pl.BlockSpec((NB2, CT), lambda i: (i, 0)),
            pl.BlockSpec((NB2, CS), lambda i: (i, 0)),
            pl.BlockSpec((CT + CS, M1), lambda i: (0, 0)),
        ],
        out_specs=[
            pl.BlockSpec((NB2, M1), lambda i: (i, 0)),
            pl.BlockSpec((2, M1), lambda i: (0, 0)),
        ],
        out_shape=[
            jax.ShapeDtypeStruct((BN, M1), jnp.float32),
            jax.ShapeDtypeStruct((2, M1), jnp.float32),
        ],
    )(feat_target.reshape(BN, CT), interp, W1t)

    cnt = jnp.float32(BN)
    a1, c1 = _bn_coeffs(stats1, g1, b1, cnt)

    x2, stats2 = pl.pallas_call(
        _mlp_kernel,
        grid=(g2n,),
        in_specs=[
            pl.BlockSpec((NB2, M1), lambda i: (i, 0)),
            pl.BlockSpec((1, M1), lambda i: (0, 0)),
            pl.BlockSpec((1, M1), lambda i: (0, 0)),
            pl.BlockSpec((M1, M2), lambda i: (0, 0)),
        ],
        out_specs=[
            pl.BlockSpec((NB2, M2), lambda i: (i, 0)),
            pl.BlockSpec((2, M2), lambda i: (0, 0)),
        ],
        out_shape=[
            jax.ShapeDtypeStruct((BN, M2), jnp.float32),
            jax.ShapeDtypeStruct((2, M2), jnp.float32),
        ],
    )(x1, a1, c1, W2.T)

    a2, c2 = _bn_coeffs(stats2, g2, b2, cnt)

    out = pl.pallas_call(
        _bnrelu_kernel,
        grid=(g2n,),
        in_specs=[
            pl.BlockSpec((NB2, M2), lambda i: (i, 0)),
            pl.BlockSpec((1, M2), lambda i: (0, 0)),
            pl.BlockSpec((1, M2), lambda i: (0, 0)),
        ],
        out_specs=pl.BlockSpec((NB2, M2), lambda i: (i, 0)),
        out_shape=jax.ShapeDtypeStruct((BN, M2), jnp.float32),
    )(x2, a2, c2)
    return out.reshape(B, N, M2)
